# Initial kernel scaffold; baseline (speedup 1.0000x reference)
import jax
import jax.numpy as jnp
from jax.experimental import pallas as pl


def kernel(x, edge_index, edge_index_upper, edge_index_lower, y, node_invalid, labeled, enc_W, enc_b, params_sup, params_low, params_up):
    raise NotImplementedError("write your pallas kernel here")


# speedup vs baseline: 9.4893x; 9.4893x over previous
import functools

import jax
import jax.numpy as jnp
from jax.experimental import pallas as pl
from jax.experimental.pallas import tpu as pltpu

T, V = 64, 25
NL, NU = 12, 13
H = 64
NC = 60
EPS = 1e-5
F32 = jnp.float32


def _build_A(edge_index, valid, N):
    src, dst = edge_index[0], edge_index[1]
    ew = valid[src] * valid[dst]
    deg = jnp.ones((N,), F32).at[dst].add(ew)
    dinv = jax.lax.rsqrt(deg)
    norm = dinv[src] * dinv[dst] * ew
    A = jnp.zeros((N, N), F32).at[dst, src].add(norm)
    i = jnp.arange(N)
    return A.at[i, i].add(dinv * dinv)


def _l1_body(lab_ref, x_ref, A_ref, Wc_ref, bc_ref, z_ref, acc_ref):
    s = pl.program_id(0)
    x = x_ref[0]
    h = x @ Wc_ref[...] + bc_ref[...]
    z = A_ref[...] @ h

    @pl.when(s == 0)
    def _():
        acc_ref[...] = jnp.zeros_like(acc_ref)

    acc_ref[0:1, :] += jnp.sum(z, axis=0, keepdims=True)
    acc_ref[1:2, :] += jnp.sum(z * z, axis=0, keepdims=True)
    z_ref[0] = z


def _mid_body(zin_ref, st_ref, g_ref, be_ref, A_ref, W_ref, b_ref, z_ref,
              acc_ref, *, SN):
    s = pl.program_id(0)
    zp = zin_ref[0]
    m = st_ref[0:1, :] / SN
    var = st_ref[1:2, :] / SN - m * m
    xn = jax.nn.relu(g_ref[...] * (zp - m) * jax.lax.rsqrt(var + EPS)
                     + be_ref[...])
    h = xn @ W_ref[...] + b_ref[...]
    z = A_ref[...] @ h

    @pl.when(s == 0)
    def _():
        acc_ref[...] = jnp.zeros_like(acc_ref)

    acc_ref[0:1, :] += jnp.sum(z, axis=0, keepdims=True)
    acc_ref[1:2, :] += jnp.sum(z * z, axis=0, keepdims=True)
    z_ref[0] = z


def _final_body(zin_ref, st_ref, g_ref, be_ref, Wf_ref, bf_ref, o_ref, *,
                SN, N):
    zp = zin_ref[0]
    m = st_ref[0:1, :] / SN
    var = st_ref[1:2, :] / SN - m * m
    xn = jax.nn.relu(g_ref[...] * (zp - m) * jax.lax.rsqrt(var + EPS)
                     + be_ref[...])
    pooled = jnp.sum(xn, axis=0, keepdims=True) * (1.0 / N)
    o_ref[0] = pooled @ Wf_ref[...] + bf_ref[...]


def _const(shape):
    return pl.BlockSpec(shape, lambda s, lab: tuple(0 for _ in shape))


def _run_branch(xin, lab, A, p, enc_W, enc_b, S, N):
    SN = float(S * N)
    Wc = enc_W @ p['W1']
    bc = (enc_b @ p['W1'] + p['b1'])[None, :]

    l1 = pl.pallas_call(
        _l1_body,
        grid_spec=pltpu.PrefetchScalarGridSpec(
            num_scalar_prefetch=1,
            grid=(S,),
            in_specs=[
                pl.BlockSpec((1, N, 3), lambda s, lab: (lab[s], 0, 0)),
                _const((N, N)),
                _const((3, H)),
                _const((1, H)),
            ],
            out_specs=[
                pl.BlockSpec((1, N, H), lambda s, lab: (s, 0, 0)),
                _const((2, H)),
            ],
        ),
        out_shape=[jax.ShapeDtypeStruct((S, N, H), F32),
                   jax.ShapeDtypeStruct((2, H), F32)],
    )
    z, st = l1(lab, xin, A, Wc, bc)

    for i in (2, 3):
        mid = pl.pallas_call(
            functools.partial(_mid_body, SN=SN),
            grid=(S,),
            in_specs=[
                pl.BlockSpec((1, N, H), lambda s: (s, 0, 0)),
                pl.BlockSpec((2, H), lambda s: (0, 0)),
                pl.BlockSpec((1, H), lambda s: (0, 0)),
                pl.BlockSpec((1, H), lambda s: (0, 0)),
                pl.BlockSpec((N, N), lambda s: (0, 0)),
                pl.BlockSpec((H, H), lambda s: (0, 0)),
                pl.BlockSpec((1, H), lambda s: (0, 0)),
            ],
            out_specs=[
                pl.BlockSpec((1, N, H), lambda s: (s, 0, 0)),
                pl.BlockSpec((2, H), lambda s: (0, 0)),
            ],
            out_shape=[jax.ShapeDtypeStruct((S, N, H), F32),
                       jax.ShapeDtypeStruct((2, H), F32)],
        )
        z, st = mid(z, st, p['g%d' % (i - 1)][None, :],
                    p['be%d' % (i - 1)][None, :], A, p['W%d' % i],
                    p['b%d' % i][None, :])

    fin = pl.pallas_call(
        functools.partial(_final_body, SN=SN, N=N),
        grid=(S,),
        in_specs=[
            pl.BlockSpec((1, N, H), lambda s: (s, 0, 0)),
            pl.BlockSpec((2, H), lambda s: (0, 0)),
            pl.BlockSpec((1, H), lambda s: (0, 0)),
            pl.BlockSpec((1, H), lambda s: (0, 0)),
            pl.BlockSpec((H, NC), lambda s: (0, 0)),
            pl.BlockSpec((1, NC), lambda s: (0, 0)),
        ],
        out_specs=pl.BlockSpec((1, 1, NC), lambda s: (s, 0, 0)),
        out_shape=jax.ShapeDtypeStruct((S, 1, NC), F32),
    )
    o = fin(z, st, p['g3'][None, :], p['be3'][None, :], p['Wf'],
            p['bf'][None, :])
    return o.reshape(S, NC)


def _loss_body(osup_ref, ol_ref, ou_ref, y_ref, lab_ref,
               ypred_ref, yl_ref, yu_ref, lsup_ref, llow_ref, lup_ref):
    def logsm(o):
        mx = jnp.max(o, axis=1, keepdims=True)
        return o - mx - jnp.log(jnp.sum(jnp.exp(o - mx), axis=1,
                                        keepdims=True))

    def amax(o):
        mx = jnp.max(o, axis=1, keepdims=True)
        io = jax.lax.broadcasted_iota(jnp.int32, o.shape, 1)
        return jnp.min(jnp.where(o >= mx, io, NC), axis=1)

    def pick_mean(olog, idx):
        io = jax.lax.broadcasted_iota(jnp.int32, olog.shape, 1)
        p = jnp.sum(jnp.where(io == idx[:, None], olog, 0.0), axis=1)
        return jnp.mean(p)

    olog = logsm(osup_ref[...])
    ollog = logsm(ol_ref[...])
    oulog = logsm(ou_ref[...])
    ypred = amax(olog)
    yl = amax(ollog)
    yu = amax(oulog)
    ypred_ref[...] = ypred[None, :]
    yl_ref[...] = yl[None, :]
    yu_ref[...] = yu[None, :]

    S = olog.shape[0]
    lab = lab_ref[...].reshape(S, 1)
    iob = jax.lax.broadcasted_iota(jnp.int32, (S, y_ref.shape[1]), 1)
    yb = jnp.broadcast_to(y_ref[...], (S, y_ref.shape[1]))
    y_lab = jnp.sum(jnp.where(iob == lab, yb, 0), axis=1)

    lsup_ref[...] = jnp.reshape(-pick_mean(olog, y_lab), (1, 1))
    llow_ref[...] = jnp.reshape(-pick_mean(ollog, yu), (1, 1))
    lup_ref[...] = jnp.reshape(-pick_mean(oulog, yl), (1, 1))


def _losses(o_sup, o_low, o_up, y, labeled):
    S = o_sup.shape[0]
    B = y.shape[0]
    out = pl.pallas_call(
        _loss_body,
        out_shape=[
            jax.ShapeDtypeStruct((1, S), jnp.int32),
            jax.ShapeDtypeStruct((1, B), jnp.int32),
            jax.ShapeDtypeStruct((1, B), jnp.int32),
            jax.ShapeDtypeStruct((1, 1), F32),
            jax.ShapeDtypeStruct((1, 1), F32),
            jax.ShapeDtypeStruct((1, 1), F32),
        ],
    )(o_sup, o_low, o_up, y[None, :], labeled[None, :])
    ypred, yl, yu, lsup, llow, lup = out
    return (ypred.reshape(S), yl.reshape(B), yu.reshape(B),
            lsup[0, 0], llow[0, 0], lup[0, 0])


def kernel(x, edge_index, edge_index_upper, edge_index_lower, y, node_invalid,
           labeled, enc_W, enc_b, params_sup, params_low, params_up):
    B = x.shape[0]
    N_full, N_low, N_up = T * V, T * NL, T * NU

    valid = jnp.logical_not(node_invalid).astype(F32)
    valid2 = valid.reshape(T, V)
    valid_low = valid2[:, :NL].reshape(-1)
    valid_up = valid2[:, NL:].reshape(-1)

    A_full = _build_A(edge_index, valid, N_full)
    A_low = _build_A(edge_index_lower, valid_low, N_low)
    A_up = _build_A(edge_index_upper, valid_up, N_up)

    x_full = x.reshape(B, N_full, 3)
    x_low = x[:, :, :NL, :].reshape(B, N_low, 3)
    x_up = x[:, :, NL:, :].reshape(B, N_up, 3)

    S = labeled.shape[0]
    ident = jnp.arange(B, dtype=jnp.int32)

    o_sup = _run_branch(x_full, labeled.astype(jnp.int32), A_full,
                        params_sup, enc_W, enc_b, S, N_full)
    o_low = _run_branch(x_low, ident, A_low, params_low, enc_W, enc_b,
                        B, N_low)
    o_up = _run_branch(x_up, ident, A_up, params_up, enc_W, enc_b,
                       B, N_up)

    return _losses(o_sup, o_low, o_up, y, labeled)


# speedup vs baseline: 20.5239x; 2.1628x over previous
import functools

import jax
import jax.numpy as jnp
from jax import lax
from jax.experimental import pallas as pl
from jax.experimental.pallas import tpu as pltpu
from jax.experimental.pallas import tpu_sc as plsc

T, V = 64, 25
NL, NU = 12, 13
H = 64
NC = 60
EPS = 1e-5
F32 = jnp.float32


def _make_sc_builder(N, E):
    NW = 32
    RPT = N // NW
    RSZ = RPT * N
    CH = E // 4
    RPTP = ((RPT + 15) // 16) * 16
    mesh = plsc.VectorSubcoreMesh(core_axis_name="c", subcore_axis_name="s")

    @functools.partial(
        pl.kernel,
        out_type=jax.ShapeDtypeStruct((N * N,), F32),
        mesh=mesh,
        compiler_params=pltpu.CompilerParams(needs_layout_passes=False),
        scratch_types=[
            pltpu.VMEM((CH,), jnp.int32),
            pltpu.VMEM((CH,), jnp.int32),
            pltpu.VMEM((N,), F32),
            pltpu.VMEM((N,), F32),
            pltpu.VMEM((N,), F32),
            pltpu.VMEM((RSZ,), F32),
        ],
    )
    def build(src_hbm, dst_hbm, valid_hbm, out_hbm, src_v, dst_v,
              valid_v, deg_v, dinv_v, A_v):
        c = lax.axis_index("c")
        s = lax.axis_index("s")
        w = c * 16 + s
        r0 = w * RPT
        zf = jnp.zeros((16,), F32)
        onef = jnp.full((16,), 1.0, F32)

        pltpu.sync_copy(valid_hbm, valid_v)

        def istep(i, _):
            o = pl.multiple_of(i * 16, 16)
            deg_v[pl.ds(o, 16)] = onef
            return 0
        lax.fori_loop(0, N // 16, istep, 0)

        def zstep(i, _):
            o = pl.multiple_of(i * 16, 16)
            A_v[pl.ds(o, 16)] = zf
            return 0
        lax.fori_loop(0, RSZ // 16, zstep, 0)

        for k in range(E // CH):
            pltpu.sync_copy(src_hbm.at[pl.ds(k * CH, CH)], src_v)
            pltpu.sync_copy(dst_hbm.at[pl.ds(k * CH, CH)], dst_v)

            def p1(i, _):
                o = pl.multiple_of(i * 16, 16)
                sv = src_v[pl.ds(o, 16)]
                dv = dst_v[pl.ds(o, 16)]
                ew = (plsc.load_gather(valid_v, [sv])
                      * plsc.load_gather(valid_v, [dv]))
                plsc.addupdate_scatter(deg_v, [dv], ew)
                return 0
            lax.fori_loop(0, CH // 16, p1, 0)

        def dstep(i, _):
            o = pl.multiple_of(i * 16, 16)
            x = deg_v[pl.ds(o, 16)]
            xi = plsc.bitcast(x, jnp.int32)
            y = plsc.bitcast(jnp.int32(0x5F3759DF)
                             - lax.shift_right_logical(xi, 1), F32)
            for _ in range(3):
                y = y * (1.5 - 0.5 * x * y * y)
            dinv_v[pl.ds(o, 16)] = y
            return 0
        lax.fori_loop(0, N // 16, dstep, 0)

        for k in range(E // CH):
            pltpu.sync_copy(src_hbm.at[pl.ds(k * CH, CH)], src_v)
            pltpu.sync_copy(dst_hbm.at[pl.ds(k * CH, CH)], dst_v)

            def p2(i, _):
                o = pl.multiple_of(i * 16, 16)
                sv = src_v[pl.ds(o, 16)]
                dv = dst_v[pl.ds(o, 16)]
                ew = (plsc.load_gather(valid_v, [sv])
                      * plsc.load_gather(valid_v, [dv]))
                nrm = (ew * plsc.load_gather(dinv_v, [sv])
                       * plsc.load_gather(dinv_v, [dv]))
                inr = jnp.logical_and(dv >= r0, dv < r0 + RPT)
                lidx = (dv - r0) * N + sv
                lidx = jnp.where(inr, lidx, 0)
                plsc.addupdate_scatter(A_v, [lidx], nrm, mask=inr)
                return 0
            lax.fori_loop(0, CH // 16, p2, 0)

        for j in range(0, RPTP, 16):
            lane = lax.iota(jnp.int32, 16) + j
            ok = lane < RPT
            rg = jnp.minimum(r0 + lane, N - 1)
            dvv = plsc.load_gather(dinv_v, [rg])
            lidx = jnp.where(ok, lane * N + rg, 0)
            plsc.addupdate_scatter(A_v, [lidx], dvv * dvv, mask=ok)

        h0 = pl.multiple_of(r0 * N, 16)
        pltpu.sync_copy(A_v, out_hbm.at[pl.ds(h0, RSZ)])

    return build


def _build_A(edge_index, valid, N):
    E = edge_index.shape[1]
    flat = _make_sc_builder(N, E)(edge_index[0], edge_index[1], valid)
    return flat.reshape(N, N)


def _l1_body(lab_ref, x_ref, A_ref, Wc_ref, bc_ref, z_ref, acc_ref):
    s = pl.program_id(0)
    x = x_ref[0]
    h = x @ Wc_ref[...] + bc_ref[...]
    z = A_ref[...] @ h

    @pl.when(s == 0)
    def _():
        acc_ref[...] = jnp.zeros_like(acc_ref)

    acc_ref[0:1, :] += jnp.sum(z, axis=0, keepdims=True)
    acc_ref[1:2, :] += jnp.sum(z * z, axis=0, keepdims=True)
    z_ref[0] = z


def _mid_body(zin_ref, st_ref, g_ref, be_ref, A_ref, W_ref, b_ref, z_ref,
              acc_ref, *, SN):
    s = pl.program_id(0)
    zp = zin_ref[0]
    m = st_ref[0:1, :] / SN
    var = st_ref[1:2, :] / SN - m * m
    xn = jax.nn.relu(g_ref[...] * (zp - m) * jax.lax.rsqrt(var + EPS)
                     + be_ref[...])
    h = xn @ W_ref[...] + b_ref[...]
    z = A_ref[...] @ h

    @pl.when(s == 0)
    def _():
        acc_ref[...] = jnp.zeros_like(acc_ref)

    acc_ref[0:1, :] += jnp.sum(z, axis=0, keepdims=True)
    acc_ref[1:2, :] += jnp.sum(z * z, axis=0, keepdims=True)
    z_ref[0] = z


def _final_body(zin_ref, st_ref, g_ref, be_ref, Wf_ref, bf_ref, o_ref, *,
                SN, N):
    zp = zin_ref[0]
    m = st_ref[0:1, :] / SN
    var = st_ref[1:2, :] / SN - m * m
    xn = jax.nn.relu(g_ref[...] * (zp - m) * jax.lax.rsqrt(var + EPS)
                     + be_ref[...])
    pooled = jnp.sum(xn, axis=0, keepdims=True) * (1.0 / N)
    o_ref[0] = pooled @ Wf_ref[...] + bf_ref[...]


def _const(shape):
    return pl.BlockSpec(shape, lambda s, lab: tuple(0 for _ in shape))


def _run_branch(xin, lab, A, p, enc_W, enc_b, S, N):
    SN = float(S * N)
    Wc = enc_W @ p['W1']
    bc = (enc_b @ p['W1'] + p['b1'])[None, :]

    l1 = pl.pallas_call(
        _l1_body,
        grid_spec=pltpu.PrefetchScalarGridSpec(
            num_scalar_prefetch=1,
            grid=(S,),
            in_specs=[
                pl.BlockSpec((1, N, 3), lambda s, lab: (lab[s], 0, 0)),
                _const((N, N)),
                _const((3, H)),
                _const((1, H)),
            ],
            out_specs=[
                pl.BlockSpec((1, N, H), lambda s, lab: (s, 0, 0)),
                _const((2, H)),
            ],
        ),
        out_shape=[jax.ShapeDtypeStruct((S, N, H), F32),
                   jax.ShapeDtypeStruct((2, H), F32)],
    )
    z, st = l1(lab, xin, A, Wc, bc)

    for i in (2, 3):
        mid = pl.pallas_call(
            functools.partial(_mid_body, SN=SN),
            grid=(S,),
            in_specs=[
                pl.BlockSpec((1, N, H), lambda s: (s, 0, 0)),
                pl.BlockSpec((2, H), lambda s: (0, 0)),
                pl.BlockSpec((1, H), lambda s: (0, 0)),
                pl.BlockSpec((1, H), lambda s: (0, 0)),
                pl.BlockSpec((N, N), lambda s: (0, 0)),
                pl.BlockSpec((H, H), lambda s: (0, 0)),
                pl.BlockSpec((1, H), lambda s: (0, 0)),
            ],
            out_specs=[
                pl.BlockSpec((1, N, H), lambda s: (s, 0, 0)),
                pl.BlockSpec((2, H), lambda s: (0, 0)),
            ],
            out_shape=[jax.ShapeDtypeStruct((S, N, H), F32),
                       jax.ShapeDtypeStruct((2, H), F32)],
        )
        z, st = mid(z, st, p['g%d' % (i - 1)][None, :],
                    p['be%d' % (i - 1)][None, :], A, p['W%d' % i],
                    p['b%d' % i][None, :])

    fin = pl.pallas_call(
        functools.partial(_final_body, SN=SN, N=N),
        grid=(S,),
        in_specs=[
            pl.BlockSpec((1, N, H), lambda s: (s, 0, 0)),
            pl.BlockSpec((2, H), lambda s: (0, 0)),
            pl.BlockSpec((1, H), lambda s: (0, 0)),
            pl.BlockSpec((1, H), lambda s: (0, 0)),
            pl.BlockSpec((H, NC), lambda s: (0, 0)),
            pl.BlockSpec((1, NC), lambda s: (0, 0)),
        ],
        out_specs=pl.BlockSpec((1, 1, NC), lambda s: (s, 0, 0)),
        out_shape=jax.ShapeDtypeStruct((S, 1, NC), F32),
    )
    o = fin(z, st, p['g3'][None, :], p['be3'][None, :], p['Wf'],
            p['bf'][None, :])
    return o.reshape(S, NC)


def _loss_body(osup_ref, ol_ref, ou_ref, y_ref, lab_ref,
               ypred_ref, yl_ref, yu_ref, lsup_ref, llow_ref, lup_ref):
    def logsm(o):
        mx = jnp.max(o, axis=1, keepdims=True)
        return o - mx - jnp.log(jnp.sum(jnp.exp(o - mx), axis=1,
                                        keepdims=True))

    def amax(o):
        mx = jnp.max(o, axis=1, keepdims=True)
        io = jax.lax.broadcasted_iota(jnp.int32, o.shape, 1)
        return jnp.min(jnp.where(o >= mx, io, NC), axis=1)

    def pick_mean(olog, idx):
        io = jax.lax.broadcasted_iota(jnp.int32, olog.shape, 1)
        p = jnp.sum(jnp.where(io == idx[:, None], olog, 0.0), axis=1)
        return jnp.mean(p)

    olog = logsm(osup_ref[...])
    ollog = logsm(ol_ref[...])
    oulog = logsm(ou_ref[...])
    ypred = amax(olog)
    yl = amax(ollog)
    yu = amax(oulog)
    ypred_ref[...] = ypred[None, :]
    yl_ref[...] = yl[None, :]
    yu_ref[...] = yu[None, :]

    S = olog.shape[0]
    lab = lab_ref[...].reshape(S, 1)
    iob = jax.lax.broadcasted_iota(jnp.int32, (S, y_ref.shape[1]), 1)
    yb = jnp.broadcast_to(y_ref[...], (S, y_ref.shape[1]))
    y_lab = jnp.sum(jnp.where(iob == lab, yb, 0), axis=1)

    lsup_ref[...] = jnp.reshape(-pick_mean(olog, y_lab), (1, 1))
    llow_ref[...] = jnp.reshape(-pick_mean(ollog, yu), (1, 1))
    lup_ref[...] = jnp.reshape(-pick_mean(oulog, yl), (1, 1))


def _losses(o_sup, o_low, o_up, y, labeled):
    S = o_sup.shape[0]
    B = y.shape[0]
    out = pl.pallas_call(
        _loss_body,
        out_shape=[
            jax.ShapeDtypeStruct((1, S), jnp.int32),
            jax.ShapeDtypeStruct((1, B), jnp.int32),
            jax.ShapeDtypeStruct((1, B), jnp.int32),
            jax.ShapeDtypeStruct((1, 1), F32),
            jax.ShapeDtypeStruct((1, 1), F32),
            jax.ShapeDtypeStruct((1, 1), F32),
        ],
    )(o_sup, o_low, o_up, y[None, :], labeled[None, :])
    ypred, yl, yu, lsup, llow, lup = out
    return (ypred.reshape(S), yl.reshape(B), yu.reshape(B),
            lsup[0, 0], llow[0, 0], lup[0, 0])


def kernel(x, edge_index, edge_index_upper, edge_index_lower, y, node_invalid,
           labeled, enc_W, enc_b, params_sup, params_low, params_up):
    B = x.shape[0]
    N_full, N_low, N_up = T * V, T * NL, T * NU

    valid = jnp.logical_not(node_invalid).astype(F32)
    valid2 = valid.reshape(T, V)
    valid_low = valid2[:, :NL].reshape(-1)
    valid_up = valid2[:, NL:].reshape(-1)

    A_full = _build_A(edge_index, valid, N_full)
    A_low = _build_A(edge_index_lower, valid_low, N_low)
    A_up = _build_A(edge_index_upper, valid_up, N_up)

    x_full = x.reshape(B, N_full, 3)
    x_low = x[:, :, :NL, :].reshape(B, N_low, 3)
    x_up = x[:, :, NL:, :].reshape(B, N_up, 3)

    S = labeled.shape[0]
    ident = jnp.arange(B, dtype=jnp.int32)

    o_sup = _run_branch(x_full, labeled.astype(jnp.int32), A_full,
                        params_sup, enc_W, enc_b, S, N_full)
    o_low = _run_branch(x_low, ident, A_low, params_low, enc_W, enc_b,
                        B, N_low)
    o_up = _run_branch(x_up, ident, A_up, params_up, enc_W, enc_b,
                       B, N_up)

    return _losses(o_sup, o_low, o_up, y, labeled)


# speedup vs baseline: 42.3149x; 2.0617x over previous
import functools

import jax
import jax.numpy as jnp
from jax import lax
from jax.experimental import pallas as pl
from jax.experimental.pallas import tpu as pltpu
from jax.experimental.pallas import tpu_sc as plsc

T, V = 64, 25
NL, NU = 12, 13
H = 64
NC = 60
EPS = 1e-5
F32 = jnp.float32


def _make_sc_builder(N, E):
    NW = 32
    RPT = N // NW
    RSZ = RPT * N
    CH = E // 4
    RPTP = ((RPT + 15) // 16) * 16
    mesh = plsc.VectorSubcoreMesh(core_axis_name="c", subcore_axis_name="s")

    @functools.partial(
        pl.kernel,
        out_type=jax.ShapeDtypeStruct((N * N,), F32),
        mesh=mesh,
        compiler_params=pltpu.CompilerParams(needs_layout_passes=False),
        scratch_types=[
            pltpu.VMEM((CH,), jnp.int32),
            pltpu.VMEM((CH,), jnp.int32),
            pltpu.VMEM((N,), F32),
            pltpu.VMEM((N,), F32),
            pltpu.VMEM((N,), F32),
            pltpu.VMEM((RSZ,), F32),
        ],
    )
    def build(src_hbm, dst_hbm, valid_hbm, out_hbm, src_v, dst_v,
              valid_v, deg_v, dinv_v, A_v):
        c = lax.axis_index("c")
        s = lax.axis_index("s")
        w = c * 16 + s
        r0 = w * RPT
        zf = jnp.zeros((16,), F32)
        onef = jnp.full((16,), 1.0, F32)

        pltpu.sync_copy(valid_hbm, valid_v)

        def istep(i, _):
            o = pl.multiple_of(i * 16, 16)
            deg_v[pl.ds(o, 16)] = onef
            return 0
        lax.fori_loop(0, N // 16, istep, 0)

        def zstep(i, _):
            o = pl.multiple_of(i * 16, 16)
            A_v[pl.ds(o, 16)] = zf
            return 0
        lax.fori_loop(0, RSZ // 16, zstep, 0)

        for k in range(E // CH):
            pltpu.sync_copy(src_hbm.at[pl.ds(k * CH, CH)], src_v)
            pltpu.sync_copy(dst_hbm.at[pl.ds(k * CH, CH)], dst_v)

            def p1(i, _):
                o = pl.multiple_of(i * 16, 16)
                sv = src_v[pl.ds(o, 16)]
                dv = dst_v[pl.ds(o, 16)]
                ew = (plsc.load_gather(valid_v, [sv])
                      * plsc.load_gather(valid_v, [dv]))
                plsc.addupdate_scatter(deg_v, [dv], ew)
                return 0
            lax.fori_loop(0, CH // 16, p1, 0)

        def dstep(i, _):
            o = pl.multiple_of(i * 16, 16)
            x = deg_v[pl.ds(o, 16)]
            xi = plsc.bitcast(x, jnp.int32)
            y = plsc.bitcast(jnp.int32(0x5F3759DF)
                             - lax.shift_right_logical(xi, 1), F32)
            for _ in range(3):
                y = y * (1.5 - 0.5 * x * y * y)
            dinv_v[pl.ds(o, 16)] = y
            return 0
        lax.fori_loop(0, N // 16, dstep, 0)

        for k in range(E // CH):
            pltpu.sync_copy(src_hbm.at[pl.ds(k * CH, CH)], src_v)
            pltpu.sync_copy(dst_hbm.at[pl.ds(k * CH, CH)], dst_v)

            def p2(i, _):
                o = pl.multiple_of(i * 16, 16)
                sv = src_v[pl.ds(o, 16)]
                dv = dst_v[pl.ds(o, 16)]
                ew = (plsc.load_gather(valid_v, [sv])
                      * plsc.load_gather(valid_v, [dv]))
                nrm = (ew * plsc.load_gather(dinv_v, [sv])
                       * plsc.load_gather(dinv_v, [dv]))
                inr = jnp.logical_and(dv >= r0, dv < r0 + RPT)
                lidx = (dv - r0) * N + sv
                lidx = jnp.where(inr, lidx, 0)
                plsc.addupdate_scatter(A_v, [lidx], nrm, mask=inr)
                return 0
            lax.fori_loop(0, CH // 16, p2, 0)

        for j in range(0, RPTP, 16):
            lane = lax.iota(jnp.int32, 16) + j
            ok = lane < RPT
            rg = jnp.minimum(r0 + lane, N - 1)
            dvv = plsc.load_gather(dinv_v, [rg])
            lidx = jnp.where(ok, lane * N + rg, 0)
            plsc.addupdate_scatter(A_v, [lidx], dvv * dvv, mask=ok)

        h0 = pl.multiple_of(r0 * N, 16)
        pltpu.sync_copy(A_v, out_hbm.at[pl.ds(h0, RSZ)])

    return build


def _build_A(edge_index, valid, N):
    E = edge_index.shape[1]
    flat = _make_sc_builder(N, E)(edge_index[0], edge_index[1], valid)
    return flat.reshape(N, N)


C = 4


def _l1p_body(lab_ref, x0_ref, x1_ref, x2_ref, x3_ref, Wc_ref, bc_ref,
              h_ref):
    Wc = Wc_ref[...]
    bc = bc_ref[...]
    hs = [x_ref[0] @ Wc + bc for x_ref in (x0_ref, x1_ref, x2_ref, x3_ref)]
    h_ref[...] = jnp.concatenate(hs, axis=1)


def _lay_body(zin_ref, st_ref, g_ref, be_ref, A_ref, W_ref, b_ref, R_ref,
              z_ref, stout_ref, acc_ref, *, SN, has_bn):
    s = pl.program_id(0)
    zp = zin_ref[...]
    if has_bn:
        m = jnp.tile(st_ref[0:1, :] / SN, (1, C))
        var = jnp.tile(st_ref[1:2, :] / SN, (1, C)) - m * m
        gt = jnp.tile(g_ref[...], (1, C))
        bt = jnp.tile(be_ref[...], (1, C))
        xn = jax.nn.relu(gt * (zp - m) * jax.lax.rsqrt(var + EPS) + bt)
        h = xn @ W_ref[...] + jnp.tile(b_ref[...], (1, C))
    else:
        h = zp
    z = A_ref[...] @ h

    @pl.when(s == 0)
    def _():
        acc_ref[...] = jnp.zeros_like(acc_ref)

    acc_ref[0:1, :] += jnp.sum(z, axis=0, keepdims=True)
    acc_ref[1:2, :] += jnp.sum(z * z, axis=0, keepdims=True)
    z_ref[...] = z

    @pl.when(s == pl.num_programs(0) - 1)
    def _():
        stout_ref[...] = acc_ref[...] @ R_ref[...]


def _fin_body(zin_ref, st_ref, g_ref, be_ref, Wfb_ref, bfb_ref, o_ref, *,
              SN, N):
    zp = zin_ref[...]
    m = jnp.tile(st_ref[0:1, :] / SN, (1, C))
    var = jnp.tile(st_ref[1:2, :] / SN, (1, C)) - m * m
    gt = jnp.tile(g_ref[...], (1, C))
    bt = jnp.tile(be_ref[...], (1, C))
    xn = jax.nn.relu(gt * (zp - m) * jax.lax.rsqrt(var + EPS) + bt)
    pooled = jnp.sum(xn, axis=0, keepdims=True) * (1.0 / N)
    o_ref[0] = pooled @ Wfb_ref[...] + bfb_ref[...]


def _run_branch(xin, lab, A, p, enc_W, enc_b, S, N):
    SN = float(S * N)
    SC_ = S // C
    Wc = enc_W @ p['W1']
    bc = (enc_b @ p['W1'] + p['b1'])[None, :]
    eyeH = jnp.eye(H, dtype=F32)
    R = jnp.tile(eyeH, (C, 1))

    def _xspec(j):
        return pl.BlockSpec((1, N, 3), lambda s, lab: (lab[C * s + j], 0, 0))

    l1p = pl.pallas_call(
        _l1p_body,
        grid_spec=pltpu.PrefetchScalarGridSpec(
            num_scalar_prefetch=1,
            grid=(SC_,),
            in_specs=[
                _xspec(0), _xspec(1), _xspec(2), _xspec(3),
                pl.BlockSpec((3, H), lambda s, lab: (0, 0)),
                pl.BlockSpec((1, H), lambda s, lab: (0, 0)),
            ],
            out_specs=pl.BlockSpec((N, C * H), lambda s, lab: (0, s)),
        ),
        out_shape=jax.ShapeDtypeStruct((N, S * H), F32),
    )
    h1 = l1p(lab, xin, xin, xin, xin, Wc, bc)

    def layer_call(zin, st, g, be, W, b, has_bn):
        args = [
            pl.BlockSpec((N, C * H), lambda s: (0, s)),
            pl.BlockSpec((2, H), lambda s: (0, 0)),
            pl.BlockSpec((1, H), lambda s: (0, 0)),
            pl.BlockSpec((1, H), lambda s: (0, 0)),
            pl.BlockSpec((N, N), lambda s: (0, 0)),
            pl.BlockSpec((C * H, C * H), lambda s: (0, 0)),
            pl.BlockSpec((1, H), lambda s: (0, 0)),
            pl.BlockSpec((C * H, H), lambda s: (0, 0)),
        ]
        lay = pl.pallas_call(
            functools.partial(_lay_body, SN=SN, has_bn=has_bn),
            grid=(SC_,),
            in_specs=args,
            out_specs=[
                pl.BlockSpec((N, C * H), lambda s: (0, s)),
                pl.BlockSpec((2, H), lambda s: (0, 0)),
            ],
            out_shape=[jax.ShapeDtypeStruct((N, S * H), F32),
                       jax.ShapeDtypeStruct((2, H), F32)],
            scratch_shapes=[pltpu.VMEM((2, C * H), F32)],
        )
        return lay(zin, st, g, be, A, W, b, R)

    zeroH = jnp.zeros((1, H), F32)
    Wblk = {}
    for i in (2, 3):
        Wblk[i] = jnp.kron(jnp.eye(C, dtype=F32), p['W%d' % i])

    z, st = layer_call(h1, jnp.ones((2, H), F32), zeroH, zeroH,
                       jnp.zeros((C * H, C * H), F32), zeroH, has_bn=False)
    z, st = layer_call(z, st, p['g1'][None, :], p['be1'][None, :],
                       Wblk[2], p['b2'][None, :], has_bn=True)
    z, st = layer_call(z, st, p['g2'][None, :], p['be2'][None, :],
                       Wblk[3], p['b3'][None, :], has_bn=True)

    Wfb = jnp.kron(jnp.eye(C, dtype=F32), p['Wf'])
    bfb = jnp.tile(p['bf'][None, :], (1, C))
    fin = pl.pallas_call(
        functools.partial(_fin_body, SN=SN, N=N),
        grid=(SC_,),
        in_specs=[
            pl.BlockSpec((N, C * H), lambda s: (0, s)),
            pl.BlockSpec((2, H), lambda s: (0, 0)),
            pl.BlockSpec((1, H), lambda s: (0, 0)),
            pl.BlockSpec((1, H), lambda s: (0, 0)),
            pl.BlockSpec((C * H, C * NC), lambda s: (0, 0)),
            pl.BlockSpec((1, C * NC), lambda s: (0, 0)),
        ],
        out_specs=pl.BlockSpec((1, 1, C * NC), lambda s: (s, 0, 0)),
        out_shape=jax.ShapeDtypeStruct((SC_, 1, C * NC), F32),
    )
    o = fin(z, st, p['g3'][None, :], p['be3'][None, :], Wfb, bfb)
    return o.reshape(S, NC)


def _loss_body(osup_ref, ol_ref, ou_ref, y_ref, lab_ref,
               ypred_ref, yl_ref, yu_ref, lsup_ref, llow_ref, lup_ref):
    def logsm(o):
        mx = jnp.max(o, axis=1, keepdims=True)
        return o - mx - jnp.log(jnp.sum(jnp.exp(o - mx), axis=1,
                                        keepdims=True))

    def amax(o):
        mx = jnp.max(o, axis=1, keepdims=True)
        io = jax.lax.broadcasted_iota(jnp.int32, o.shape, 1)
        return jnp.min(jnp.where(o >= mx, io, NC), axis=1)

    def pick_mean(olog, idx):
        io = jax.lax.broadcasted_iota(jnp.int32, olog.shape, 1)
        p = jnp.sum(jnp.where(io == idx[:, None], olog, 0.0), axis=1)
        return jnp.mean(p)

    olog = logsm(osup_ref[...])
    ollog = logsm(ol_ref[...])
    oulog = logsm(ou_ref[...])
    ypred = amax(olog)
    yl = amax(ollog)
    yu = amax(oulog)
    ypred_ref[...] = ypred[None, :]
    yl_ref[...] = yl[None, :]
    yu_ref[...] = yu[None, :]

    S = olog.shape[0]
    lab = lab_ref[...].reshape(S, 1)
    iob = jax.lax.broadcasted_iota(jnp.int32, (S, y_ref.shape[1]), 1)
    yb = jnp.broadcast_to(y_ref[...], (S, y_ref.shape[1]))
    y_lab = jnp.sum(jnp.where(iob == lab, yb, 0), axis=1)

    lsup_ref[...] = jnp.reshape(-pick_mean(olog, y_lab), (1, 1))
    llow_ref[...] = jnp.reshape(-pick_mean(ollog, yu), (1, 1))
    lup_ref[...] = jnp.reshape(-pick_mean(oulog, yl), (1, 1))


def _losses(o_sup, o_low, o_up, y, labeled):
    S = o_sup.shape[0]
    B = y.shape[0]
    out = pl.pallas_call(
        _loss_body,
        out_shape=[
            jax.ShapeDtypeStruct((1, S), jnp.int32),
            jax.ShapeDtypeStruct((1, B), jnp.int32),
            jax.ShapeDtypeStruct((1, B), jnp.int32),
            jax.ShapeDtypeStruct((1, 1), F32),
            jax.ShapeDtypeStruct((1, 1), F32),
            jax.ShapeDtypeStruct((1, 1), F32),
        ],
    )(o_sup, o_low, o_up, y[None, :], labeled[None, :])
    ypred, yl, yu, lsup, llow, lup = out
    return (ypred.reshape(S), yl.reshape(B), yu.reshape(B),
            lsup[0, 0], llow[0, 0], lup[0, 0])


def kernel(x, edge_index, edge_index_upper, edge_index_lower, y, node_invalid,
           labeled, enc_W, enc_b, params_sup, params_low, params_up):
    B = x.shape[0]
    N_full, N_low, N_up = T * V, T * NL, T * NU

    valid = jnp.logical_not(node_invalid).astype(F32)
    valid2 = valid.reshape(T, V)
    valid_low = valid2[:, :NL].reshape(-1)
    valid_up = valid2[:, NL:].reshape(-1)

    A_full = _build_A(edge_index, valid, N_full)
    A_low = _build_A(edge_index_lower, valid_low, N_low)
    A_up = _build_A(edge_index_upper, valid_up, N_up)

    x_full = x.reshape(B, N_full, 3)
    x_low = x[:, :, :NL, :].reshape(B, N_low, 3)
    x_up = x[:, :, NL:, :].reshape(B, N_up, 3)

    S = labeled.shape[0]
    ident = jnp.arange(B, dtype=jnp.int32)

    o_sup = _run_branch(x_full, labeled.astype(jnp.int32), A_full,
                        params_sup, enc_W, enc_b, S, N_full)
    o_low = _run_branch(x_low, ident, A_low, params_low, enc_W, enc_b,
                        B, N_low)
    o_up = _run_branch(x_up, ident, A_up, params_up, enc_W, enc_b,
                       B, N_up)

    return _losses(o_sup, o_low, o_up, y, labeled)
